# Initial kernel scaffold; baseline (speedup 1.0000x reference)
import jax
import jax.numpy as jnp
from jax.experimental import pallas as pl


def kernel(body_pose, jaw_pose, leye_pose, reye_pose, lhand_pose, rhand_pose, expr, tri_feat, scale_param, rgb_param, mean_3d, expr_dirs, skinning_weight, joint_zero_pose, transform_mat_neutral_pose, mesh_neutral_pose_wo_upsample, parents, lr_to_hr, is_rhand, is_lhand, is_face, W_pose, W1, b1, W_mo, W_so):
    raise NotImplementedError("write your pallas kernel here")


# speedup vs baseline: 1.3131x; 1.3131x over previous
import functools

import jax
import jax.numpy as jnp
from jax import lax
from jax.experimental import pallas as pl
from jax.experimental.pallas import tpu as pltpu
from jax.experimental.pallas import tpu_sc as plsc

N = 100000
M = 10475
J = 55
EXPR = 50
C = 32
H = 64
POSE_DIM = 63 + 3 + 3 + 3 + 45 + 45 + EXPR

BQ = 800
MP = 10752
MB = 1536
NCH = MP // MB
PAD_VAL = 1e17

NW = 32
CHUNK_ROWS = 128
TOT_CH = 782
NPAD_OUT = TOT_CH * CHUNK_ROWS
BASE_CH = TOT_CH // NW
EXTRA_CH = TOT_CH - BASE_CH * NW
LR_PAD = 10480
DPAD = 128


def _tc_body(pose_ref, wpose_ref, e_ref, w1_ref, b1_ref, wmo_ref, wso_ref,
             refs_ref, tri_ref, sp_ref, rgb_ref, m3_ref, dirs_ref,
             isr_ref, isl_ref, isf_ref,
             mo_out, sc_out, rgb_out, mr_out, sr_out, nnj_out, msk_out):
    f32 = jnp.float32
    hi = lax.Precision.HIGHEST
    pose_emb = jnp.tanh(jnp.dot(pose_ref[...], wpose_ref[...],
                                precision=hi, preferred_element_type=f32))
    h = jnp.maximum(
        jnp.dot(tri_ref[...] + pose_emb, w1_ref[...],
                precision=hi, preferred_element_type=f32) + b1_ref[...], 0.0)
    mo = jnp.dot(h, wmo_ref[...], precision=hi, preferred_element_type=f32)
    so = jnp.dot(h, wso_ref[...], precision=hi, preferred_element_type=f32)
    sp = sp_ref[...]
    sc_out[...] = jnp.broadcast_to(jnp.exp(sp), (BQ, 3))
    sr_out[...] = jnp.broadcast_to(jnp.exp(sp + so), (BQ, 3))
    rgb_out[...] = (jnp.tanh(rgb_ref[...]) + 1.0) / 2.0
    eo = jnp.dot(dirs_ref[...], e_ref[...],
                 precision=hi, preferred_element_type=f32)
    m3 = m3_ref[...]
    q = m3 + eo
    mo_out[...] = q
    mr_out[...] = (m3 + mo) + eo
    msk_out[...] = ((isr_ref[...] + isl_ref[...] + isf_ref[...]) > 0
                    ).astype(jnp.int32)
    qx = q[:, 0:1]
    qy = q[:, 1:2]
    qz = q[:, 2:3]

    def body(c, carry):
        bd, bc = carry
        r = refs_ref[c]
        dx = qx - r[0:1, :]
        dy = qy - r[1:2, :]
        dz = qz - r[2:3, :]
        d = (dx * dx + dy * dy) + dz * dz
        upd = d < bd
        return jnp.where(upd, d, bd), jnp.where(upd, c, bc)

    bd0 = jnp.full((BQ, MB), jnp.inf, f32)
    bc0 = jnp.zeros((BQ, MB), jnp.int32)
    bd, bc = lax.fori_loop(0, NCH, body, (bd0, bc0))
    dmin = jnp.min(bd, axis=1, keepdims=True)
    lane = lax.broadcasted_iota(jnp.int32, (BQ, MB), 1)
    jcand = jnp.where(bd == dmin, bc * MB + lane, jnp.int32(2147483647))
    nnj_out[...] = jnp.min(jcand, axis=1, keepdims=True)


def _tc_call(pose2d, W_pose, E, W1, b1_2d, W_mo, W_so, refs_c,
             tri_feat, scale_param, rgb_param, mean_3d, dirs, isr, isl, isf):
    f32 = jnp.float32
    i32 = jnp.int32
    g = N // BQ
    const = lambda *shape: pl.BlockSpec(shape, lambda i: (0,) * len(shape))
    row = lambda d: pl.BlockSpec((BQ, d), lambda i: (i, 0))
    return pl.pallas_call(
        _tc_body,
        grid=(g,),
        in_specs=[
            const(1, POSE_DIM),
            const(POSE_DIM, C),
            const(3 * EXPR, 3),
            const(C, H),
            const(1, H),
            const(H, 3),
            const(H, 1),
            const(NCH, 3, MB),
            row(C),
            row(1),
            row(3),
            row(3),
            row(3 * EXPR),
            row(1),
            row(1),
            row(1),
        ],
        out_specs=[row(3), row(3), row(3), row(3), row(3), row(1), row(1)],
        out_shape=[
            jax.ShapeDtypeStruct((N, 3), f32),
            jax.ShapeDtypeStruct((N, 3), f32),
            jax.ShapeDtypeStruct((N, 3), f32),
            jax.ShapeDtypeStruct((N, 3), f32),
            jax.ShapeDtypeStruct((N, 3), f32),
            jax.ShapeDtypeStruct((N, 1), i32),
            jax.ShapeDtypeStruct((N, 1), i32),
        ],
    )(pose2d, W_pose, E, W1, b1_2d, W_mo, W_so, refs_c,
      tri_feat, scale_param, rgb_param, mean_3d, dirs, isr, isl, isf)


def _sc_gather(knn_flat, msk_flat, lr_pad, table_pad):
    mesh = plsc.VectorSubcoreMesh(core_axis_name="c", subcore_axis_name="s")

    @functools.partial(
        pl.kernel,
        mesh=mesh,
        out_type=jax.ShapeDtypeStruct((NPAD_OUT, DPAD), jnp.float32),
        scratch_types=[
            pltpu.VMEM((CHUNK_ROWS,), jnp.int32),
            pltpu.VMEM((CHUNK_ROWS,), jnp.int32),
            pltpu.VMEM((CHUNK_ROWS,), jnp.int32),
            pltpu.VMEM((CHUNK_ROWS,), jnp.int32),
            pltpu.VMEM((CHUNK_ROWS, DPAD), jnp.float32),
            pltpu.SemaphoreType.DMA,
        ],
    )
    def _sc(knn_hbm, msk_hbm, lr_hbm, tab_hbm, out_hbm,
            knnv, hrv, mskv, selv, rowsv, sem):
        wid = lax.axis_index("s") * 2 + lax.axis_index("c")
        nch = jnp.where(wid < EXTRA_CH, BASE_CH + 1, BASE_CH)
        cbase = wid * BASE_CH + jnp.minimum(wid, EXTRA_CH)

        def chunk_body(k, carry):
            row0 = (cbase + k) * CHUNK_ROWS
            pltpu.sync_copy(knn_hbm.at[pl.ds(row0, CHUNK_ROWS)], knnv)
            pltpu.sync_copy(msk_hbm.at[pl.ds(row0, CHUNK_ROWS)], mskv)
            pltpu.async_copy(lr_hbm.at[knnv], hrv, sem).wait()
            for v in range(CHUNK_ROWS // 16):
                sl = pl.ds(v * 16, 16)
                gi = lax.broadcasted_iota(jnp.int32, (16,), 0) + (row0 + v * 16)
                selv[sl] = jnp.where(mskv[sl] > 0, gi, hrv[sl])
            pltpu.async_copy(tab_hbm.at[selv], rowsv, sem).wait()
            pltpu.sync_copy(rowsv, out_hbm.at[pl.ds(row0, CHUNK_ROWS)])
            return carry

        lax.fori_loop(0, nch, chunk_body, 0)

    return _sc(knn_flat, msk_flat, lr_pad, table_pad)


def kernel(body_pose, jaw_pose, leye_pose, reye_pose, lhand_pose, rhand_pose,
           expr, tri_feat, scale_param, rgb_param, mean_3d, expr_dirs,
           skinning_weight, joint_zero_pose, transform_mat_neutral_pose,
           mesh_neutral_pose_wo_upsample, parents, lr_to_hr,
           is_rhand, is_lhand, is_face, W_pose, W1, b1, W_mo, W_so):
    f32 = jnp.float32
    pose2d = jnp.concatenate([body_pose, jaw_pose, leye_pose, reye_pose,
                              lhand_pose, rhand_pose, expr])[None, :]
    ec = expr[:, None].astype(f32)
    z = jnp.zeros((EXPR, 1), f32)
    E = jnp.concatenate([
        jnp.concatenate([ec, z, z], axis=1),
        jnp.concatenate([z, ec, z], axis=1),
        jnp.concatenate([z, z, ec], axis=1),
    ], axis=0)
    refs_p = jnp.pad(mesh_neutral_pose_wo_upsample, ((0, MP - M), (0, 0)),
                     constant_values=PAD_VAL)
    refs_c = refs_p.T.reshape(3, NCH, MB).transpose(1, 0, 2)
    dirs = expr_dirs.reshape(N, 3 * EXPR)
    isr = is_rhand.astype(jnp.int32)[:, None]
    isl = is_lhand.astype(jnp.int32)[:, None]
    isf = is_face.astype(jnp.int32)[:, None]

    (mean_out, scale, rgb, mean_refined, scale_refined, nnj, msk) = _tc_call(
        pose2d, W_pose, E, W1, b1[None, :], W_mo, W_so, refs_c,
        tri_feat, scale_param, rgb_param, mean_3d, dirs, isr, isl, isf)

    knn_flat = jnp.pad(nnj[:, 0], (0, NPAD_OUT - N))
    msk_flat = jnp.pad(msk[:, 0], (0, NPAD_OUT - N))
    lr_pad = jnp.pad(lr_to_hr.astype(jnp.int32), (0, LR_PAD - M))
    tab = jnp.pad(skinning_weight, ((0, 0), (0, DPAD - J)))
    skin = _sc_gather(knn_flat, msk_flat, lr_pad, tab)[:N, :J]

    opacity = jnp.ones((N, 1), f32)
    rotation = jnp.tile(jnp.array([1.0, 0.0, 0.0, 0.0], f32)[None, :], (N, 1))
    return (mean_out, opacity, scale, rotation, rgb, mean_refined,
            scale_refined, joint_zero_pose, transform_mat_neutral_pose,
            parents, skin)


# speedup vs baseline: 1.5646x; 1.1915x over previous
import functools

import jax
import jax.numpy as jnp
from jax import lax
from jax.experimental import pallas as pl
from jax.experimental.pallas import tpu as pltpu
from jax.experimental.pallas import tpu_sc as plsc

N = 100000
M = 10475
J = 55
EXPR = 50
C = 32
H = 64
POSE_DIM = 63 + 3 + 3 + 3 + 45 + 45 + EXPR

BQ = 800
MP = 10752
MB = 1536
NCH = MP // MB
PAD_VAL = 1e17

NW = 32
CHUNK_ROWS = 128
TOT_CH = 782
NPAD_OUT = TOT_CH * CHUNK_ROWS
BASE_CH = TOT_CH // NW
EXTRA_CH = TOT_CH - BASE_CH * NW
LR_PAD = 10480
DPAD = 128


def _tc_body(pose_ref, wpose_ref, e_ref, w1_ref, b1_ref, wmo_ref, wso_ref,
             refs_ref, tri_ref, sp_ref, rgb_ref, m3_ref, dirs_ref,
             isr_ref, isl_ref, isf_ref,
             mo_out, sc_out, rgb_out, mr_out, sr_out, nnj_out, msk_out):
    f32 = jnp.float32
    hi = lax.Precision.HIGHEST
    pose_emb = jnp.tanh(jnp.dot(pose_ref[...], wpose_ref[...],
                                precision=hi, preferred_element_type=f32))
    h = jnp.maximum(
        jnp.dot(tri_ref[...] + pose_emb, w1_ref[...],
                preferred_element_type=f32) + b1_ref[...], 0.0)
    mo = jnp.dot(h, wmo_ref[...], preferred_element_type=f32)
    so = jnp.dot(h, wso_ref[...], preferred_element_type=f32)
    sp = sp_ref[...]
    sc_out[...] = jnp.broadcast_to(jnp.exp(sp), (BQ, 3))
    sr_out[...] = jnp.broadcast_to(jnp.exp(sp + so), (BQ, 3))
    rgb_out[...] = (jnp.tanh(rgb_ref[...]) + 1.0) / 2.0
    eo = jnp.dot(dirs_ref[...], e_ref[...],
                 precision=hi, preferred_element_type=f32)
    m3 = m3_ref[...]
    q = m3 + eo
    mo_out[...] = q
    mr_out[...] = (m3 + mo) + eo
    msk_out[...] = ((isr_ref[...] + isl_ref[...] + isf_ref[...]) > 0
                    ).astype(jnp.int32)
    qx = q[:, 0:1]
    qy = q[:, 1:2]
    qz = q[:, 2:3]

    def body(c, carry):
        r = refs_ref[c]
        dx = qx - r[0:1, :]
        dy = qy - r[1:2, :]
        dz = qz - r[2:3, :]
        d = (dx * dx + dy * dy) + dz * dz
        di = lax.bitcast_convert_type(d, jnp.int32)
        key = (di & jnp.int32(-8)) | c
        return jnp.minimum(carry, key)

    bk0 = jnp.full((BQ, MB), jnp.int32(2147483647))
    bk = lax.fori_loop(0, NCH, body, bk0)
    kmin = jnp.min(bk, axis=1, keepdims=True)
    lane = lax.broadcasted_iota(jnp.int32, (BQ, MB), 1)
    lmin = jnp.min(jnp.where(bk == kmin, lane, jnp.int32(2147483647)),
                   axis=1, keepdims=True)
    nnj_out[...] = (kmin & 7) * MB + lmin


def _tc_call(pose2d, W_pose, E, W1, b1_2d, W_mo, W_so, refs_c,
             tri_feat, scale_param, rgb_param, mean_3d, dirs, isr, isl, isf):
    f32 = jnp.float32
    i32 = jnp.int32
    g = N // BQ
    const = lambda *shape: pl.BlockSpec(shape, lambda i: (0,) * len(shape))
    row = lambda d: pl.BlockSpec((BQ, d), lambda i: (i, 0))
    return pl.pallas_call(
        _tc_body,
        grid=(g,),
        in_specs=[
            const(1, POSE_DIM),
            const(POSE_DIM, C),
            const(3 * EXPR, 3),
            const(C, H),
            const(1, H),
            const(H, 3),
            const(H, 1),
            const(NCH, 3, MB),
            row(C),
            row(1),
            row(3),
            row(3),
            row(3 * EXPR),
            row(1),
            row(1),
            row(1),
        ],
        out_specs=[row(3), row(3), row(3), row(3), row(3), row(1), row(1)],
        out_shape=[
            jax.ShapeDtypeStruct((N, 3), f32),
            jax.ShapeDtypeStruct((N, 3), f32),
            jax.ShapeDtypeStruct((N, 3), f32),
            jax.ShapeDtypeStruct((N, 3), f32),
            jax.ShapeDtypeStruct((N, 3), f32),
            jax.ShapeDtypeStruct((N, 1), i32),
            jax.ShapeDtypeStruct((N, 1), i32),
        ],
    )(pose2d, W_pose, E, W1, b1_2d, W_mo, W_so, refs_c,
      tri_feat, scale_param, rgb_param, mean_3d, dirs, isr, isl, isf)


def _sc_gather(knn_flat, msk_flat, lr_pad, table_pad):
    mesh = plsc.VectorSubcoreMesh(core_axis_name="c", subcore_axis_name="s")

    @functools.partial(
        pl.kernel,
        mesh=mesh,
        out_type=jax.ShapeDtypeStruct((NPAD_OUT, DPAD), jnp.float32),
        scratch_types=[
            pltpu.VMEM((CHUNK_ROWS,), jnp.int32),
            pltpu.VMEM((CHUNK_ROWS,), jnp.int32),
            pltpu.VMEM((CHUNK_ROWS,), jnp.int32),
            pltpu.VMEM((CHUNK_ROWS,), jnp.int32),
            pltpu.VMEM((CHUNK_ROWS, DPAD), jnp.float32),
            pltpu.SemaphoreType.DMA,
        ],
    )
    def _sc(knn_hbm, msk_hbm, lr_hbm, tab_hbm, out_hbm,
            knnv, hrv, mskv, selv, rowsv, sem):
        wid = lax.axis_index("s") * 2 + lax.axis_index("c")
        nch = jnp.where(wid < EXTRA_CH, BASE_CH + 1, BASE_CH)
        cbase = wid * BASE_CH + jnp.minimum(wid, EXTRA_CH)

        def chunk_body(k, carry):
            row0 = (cbase + k) * CHUNK_ROWS
            pltpu.sync_copy(knn_hbm.at[pl.ds(row0, CHUNK_ROWS)], knnv)
            pltpu.sync_copy(msk_hbm.at[pl.ds(row0, CHUNK_ROWS)], mskv)
            pltpu.async_copy(lr_hbm.at[knnv], hrv, sem).wait()
            for v in range(CHUNK_ROWS // 16):
                sl = pl.ds(v * 16, 16)
                gi = lax.broadcasted_iota(jnp.int32, (16,), 0) + (row0 + v * 16)
                selv[sl] = jnp.where(mskv[sl] > 0, gi, hrv[sl])
            pltpu.async_copy(tab_hbm.at[selv], rowsv, sem).wait()
            pltpu.sync_copy(rowsv, out_hbm.at[pl.ds(row0, CHUNK_ROWS)])
            return carry

        lax.fori_loop(0, nch, chunk_body, 0)

    return _sc(knn_flat, msk_flat, lr_pad, table_pad)


def kernel(body_pose, jaw_pose, leye_pose, reye_pose, lhand_pose, rhand_pose,
           expr, tri_feat, scale_param, rgb_param, mean_3d, expr_dirs,
           skinning_weight, joint_zero_pose, transform_mat_neutral_pose,
           mesh_neutral_pose_wo_upsample, parents, lr_to_hr,
           is_rhand, is_lhand, is_face, W_pose, W1, b1, W_mo, W_so):
    f32 = jnp.float32
    pose2d = jnp.concatenate([body_pose, jaw_pose, leye_pose, reye_pose,
                              lhand_pose, rhand_pose, expr])[None, :]
    ec = expr[:, None].astype(f32)
    z = jnp.zeros((EXPR, 1), f32)
    E = jnp.concatenate([
        jnp.concatenate([ec, z, z], axis=1),
        jnp.concatenate([z, ec, z], axis=1),
        jnp.concatenate([z, z, ec], axis=1),
    ], axis=0)
    refs_p = jnp.pad(mesh_neutral_pose_wo_upsample, ((0, MP - M), (0, 0)),
                     constant_values=PAD_VAL)
    refs_c = refs_p.T.reshape(3, NCH, MB).transpose(1, 0, 2)
    dirs = expr_dirs.reshape(N, 3 * EXPR)
    isr = is_rhand.astype(jnp.int32)[:, None]
    isl = is_lhand.astype(jnp.int32)[:, None]
    isf = is_face.astype(jnp.int32)[:, None]

    (mean_out, scale, rgb, mean_refined, scale_refined, nnj, msk) = _tc_call(
        pose2d, W_pose, E, W1, b1[None, :], W_mo, W_so, refs_c,
        tri_feat, scale_param, rgb_param, mean_3d, dirs, isr, isl, isf)

    knn_flat = jnp.pad(nnj[:, 0], (0, NPAD_OUT - N))
    msk_flat = jnp.pad(msk[:, 0], (0, NPAD_OUT - N))
    lr_pad = jnp.pad(lr_to_hr.astype(jnp.int32), (0, LR_PAD - M))
    tab = jnp.pad(skinning_weight, ((0, 0), (0, DPAD - J)))
    skin = _sc_gather(knn_flat, msk_flat, lr_pad, tab)[:N, :J]

    opacity = jnp.ones((N, 1), f32)
    rotation = jnp.tile(jnp.array([1.0, 0.0, 0.0, 0.0], f32)[None, :], (N, 1))
    return (mean_out, opacity, scale, rotation, rgb, mean_refined,
            scale_refined, joint_zero_pose, transform_mat_neutral_pose,
            parents, skin)


# speedup vs baseline: 2.1247x; 1.3579x over previous
import functools

import jax
import jax.numpy as jnp
from jax import lax
from jax.experimental import pallas as pl
from jax.experimental.pallas import tpu as pltpu
from jax.experimental.pallas import tpu_sc as plsc

N = 100000
M = 10475
J = 55
EXPR = 50
C = 32
H = 64
POSE_DIM = 63 + 3 + 3 + 3 + 45 + 45 + EXPR

BQ = 800
MP = 10496
MB = 1536
NCH = 7
PAD_VAL = 1e17

NW = 32
CHUNK_ROWS = 128
TOT_CH = 782
NPAD_OUT = TOT_CH * CHUNK_ROWS
BASE_CH = TOT_CH // NW
EXTRA_CH = TOT_CH - BASE_CH * NW
LR_PAD = 10480
DPAD = 128


def _tc_body(pose_ref, wpose_ref, e_ref, w1_ref, b1_ref, wmo_ref, wso_ref,
             refs_ref, tri_ref, sp_ref, rgb_ref, m3_ref, dirs_ref,
             isr_ref, isl_ref, isf_ref,
             mo_out, sc_out, rgb_out, mr_out, sr_out, nnj_out):
    f32 = jnp.float32
    hi = lax.Precision.HIGHEST
    pose_emb = jnp.tanh(jnp.dot(pose_ref[...], wpose_ref[...],
                                precision=hi, preferred_element_type=f32))
    h = jnp.maximum(
        jnp.dot(tri_ref[...] + pose_emb, w1_ref[...],
                preferred_element_type=f32) + b1_ref[...], 0.0)
    mo = jnp.dot(h, wmo_ref[...], preferred_element_type=f32)
    so = jnp.dot(h, wso_ref[...], preferred_element_type=f32)
    sp = sp_ref[...]
    sc_out[...] = jnp.broadcast_to(jnp.exp(sp), (BQ, 3))
    sr_out[...] = jnp.broadcast_to(jnp.exp(sp + so), (BQ, 3))
    rgb_out[...] = (jnp.tanh(rgb_ref[...]) + 1.0) / 2.0
    eo = jnp.dot(dirs_ref[...], e_ref[...],
                 precision=hi, preferred_element_type=f32)
    m3 = m3_ref[...]
    q = m3 + eo
    mo_out[...] = q
    mr_out[...] = (m3 + mo) + eo
    msk = (isr_ref[...] + isl_ref[...] + isf_ref[...]) > 0
    qx = q[:, 0:1]
    qy = q[:, 1:2]
    qz = q[:, 2:3]

    MBL = MP - MB * (NCH - 1)
    bk = None
    bk2 = None
    for c in range(NCH):
        w = MB if c < NCH - 1 else MBL
        r = refs_ref[:, c * MB:c * MB + w]
        dx = qx - r[0:1, :]
        dy = qy - r[1:2, :]
        dz = qz - r[2:3, :]
        d = (dx * dx + dy * dy) + dz * dz
        di = lax.bitcast_convert_type(d, jnp.int32)
        keyf = lax.bitcast_convert_type((di & jnp.int32(-8)) | c, f32)
        if c < NCH - 1:
            bk = keyf if bk is None else jnp.minimum(bk, keyf)
        else:
            bk2 = keyf
    kmin = jnp.minimum(jnp.min(bk, axis=1, keepdims=True),
                       jnp.min(bk2, axis=1, keepdims=True))
    big = jnp.float32(3.0e9)
    lane1 = lax.broadcasted_iota(jnp.int32, (BQ, MB), 1).astype(f32)
    lane2 = lax.broadcasted_iota(jnp.int32, (BQ, MBL), 1).astype(f32)
    l1 = jnp.min(jnp.where(bk == kmin, lane1, big), axis=1, keepdims=True)
    l2 = jnp.min(jnp.where(bk2 == kmin, lane2, big), axis=1, keepdims=True)
    lmin = jnp.minimum(l1, l2).astype(jnp.int32)
    ki = lax.bitcast_convert_type(kmin, jnp.int32)
    j = (ki & 7) * MB + lmin
    gi = (lax.broadcasted_iota(jnp.int32, (BQ, 1), 0)
          + pl.program_id(0) * BQ)
    nnj_out[...] = jnp.where(msk, ~gi, j)


def _tc_call(pose2d, W_pose, E, W1, b1_2d, W_mo, W_so, refs_c,
             tri_feat, scale_param, rgb_param, mean_3d, dirs, isr, isl, isf):
    f32 = jnp.float32
    i32 = jnp.int32
    g = N // BQ
    const = lambda *shape: pl.BlockSpec(shape, lambda i: (0,) * len(shape))
    row = lambda d: pl.BlockSpec((BQ, d), lambda i: (i, 0))
    return pl.pallas_call(
        _tc_body,
        grid=(g,),
        in_specs=[
            const(1, POSE_DIM),
            const(POSE_DIM, C),
            const(3 * EXPR, 3),
            const(C, H),
            const(1, H),
            const(H, 3),
            const(H, 1),
            const(3, MP),
            row(C),
            row(1),
            row(3),
            row(3),
            row(3 * EXPR),
            row(1),
            row(1),
            row(1),
        ],
        out_specs=[row(3), row(3), row(3), row(3), row(3), row(1)],
        out_shape=[
            jax.ShapeDtypeStruct((N, 3), f32),
            jax.ShapeDtypeStruct((N, 3), f32),
            jax.ShapeDtypeStruct((N, 3), f32),
            jax.ShapeDtypeStruct((N, 3), f32),
            jax.ShapeDtypeStruct((N, 3), f32),
            jax.ShapeDtypeStruct((N, 1), i32),
        ],
    )(pose2d, W_pose, E, W1, b1_2d, W_mo, W_so, refs_c,
      tri_feat, scale_param, rgb_param, mean_3d, dirs, isr, isl, isf)


def _sc_gather(knn_flat, lr_pad, table_pad):
    mesh = plsc.VectorSubcoreMesh(core_axis_name="c", subcore_axis_name="s")

    @functools.partial(
        pl.kernel,
        mesh=mesh,
        out_type=jax.ShapeDtypeStruct((NPAD_OUT, DPAD), jnp.float32),
        scratch_types=[
            pltpu.VMEM((CHUNK_ROWS,), jnp.int32),
            pltpu.VMEM((CHUNK_ROWS,), jnp.int32),
            pltpu.VMEM((CHUNK_ROWS,), jnp.int32),
            pltpu.VMEM((CHUNK_ROWS,), jnp.int32),
            pltpu.VMEM((CHUNK_ROWS, DPAD), jnp.float32),
            pltpu.SemaphoreType.DMA,
        ],
    )
    def _sc(knn_hbm, lr_hbm, tab_hbm, out_hbm,
            knnv, hrv, gidxv, selv, rowsv, sem):
        wid = lax.axis_index("s") * 2 + lax.axis_index("c")
        nch = jnp.where(wid < EXTRA_CH, BASE_CH + 1, BASE_CH)
        cbase = wid * BASE_CH + jnp.minimum(wid, EXTRA_CH)

        def chunk_body(k, carry):
            row0 = (cbase + k) * CHUNK_ROWS
            pltpu.sync_copy(knn_hbm.at[pl.ds(row0, CHUNK_ROWS)], knnv)
            for v in range(CHUNK_ROWS // 16):
                sl = pl.ds(v * 16, 16)
                gidxv[sl] = jnp.maximum(knnv[sl], 0)
            pltpu.async_copy(lr_hbm.at[gidxv], hrv, sem).wait()
            for v in range(CHUNK_ROWS // 16):
                sl = pl.ds(v * 16, 16)
                p = knnv[sl]
                selv[sl] = jnp.where(p < 0, ~p, hrv[sl])
            pltpu.async_copy(tab_hbm.at[selv], rowsv, sem).wait()
            pltpu.sync_copy(rowsv, out_hbm.at[pl.ds(row0, CHUNK_ROWS)])
            return carry

        lax.fori_loop(0, nch, chunk_body, 0)

    return _sc(knn_flat, lr_pad, table_pad)


def kernel(body_pose, jaw_pose, leye_pose, reye_pose, lhand_pose, rhand_pose,
           expr, tri_feat, scale_param, rgb_param, mean_3d, expr_dirs,
           skinning_weight, joint_zero_pose, transform_mat_neutral_pose,
           mesh_neutral_pose_wo_upsample, parents, lr_to_hr,
           is_rhand, is_lhand, is_face, W_pose, W1, b1, W_mo, W_so):
    f32 = jnp.float32
    pose2d = jnp.concatenate([body_pose, jaw_pose, leye_pose, reye_pose,
                              lhand_pose, rhand_pose, expr])[None, :]
    ec = expr[:, None].astype(f32)
    z = jnp.zeros((EXPR, 1), f32)
    E = jnp.concatenate([
        jnp.concatenate([ec, z, z], axis=1),
        jnp.concatenate([z, ec, z], axis=1),
        jnp.concatenate([z, z, ec], axis=1),
    ], axis=0)
    refs_p = jnp.pad(mesh_neutral_pose_wo_upsample, ((0, MP - M), (0, 0)),
                     constant_values=PAD_VAL)
    refs_c = refs_p.T
    dirs = expr_dirs.reshape(N, 3 * EXPR)
    isr = is_rhand.astype(jnp.int32)[:, None]
    isl = is_lhand.astype(jnp.int32)[:, None]
    isf = is_face.astype(jnp.int32)[:, None]

    (mean_out, scale, rgb, mean_refined, scale_refined, nnj) = _tc_call(
        pose2d, W_pose, E, W1, b1[None, :], W_mo, W_so, refs_c,
        tri_feat, scale_param, rgb_param, mean_3d, dirs, isr, isl, isf)

    knn_flat = jnp.pad(nnj[:, 0], (0, NPAD_OUT - N))
    lr_pad = jnp.pad(lr_to_hr.astype(jnp.int32), (0, LR_PAD - M))
    tab = jnp.pad(skinning_weight, ((0, 0), (0, DPAD - J)))
    skin = _sc_gather(knn_flat, lr_pad, tab)[:N, :J]

    opacity = jnp.ones((N, 1), f32)
    rotation = jnp.tile(jnp.array([1.0, 0.0, 0.0, 0.0], f32)[None, :], (N, 1))
    return (mean_out, opacity, scale, rotation, rgb, mean_refined,
            scale_refined, joint_zero_pose, transform_mat_neutral_pose,
            parents, skin)
